# Initial kernel scaffold; baseline (speedup 1.0000x reference)
import jax
import jax.numpy as jnp
from jax.experimental import pallas as pl


def kernel(x, emb):
    raise NotImplementedError("write your pallas kernel here")


# speedup vs baseline: 4.3943x; 4.3943x over previous
import jax
import jax.numpy as jnp
from jax.experimental import pallas as pl

_BLOCK_ROWS = 1024


def _copy_body(emb_ref, o_ref):
    o_ref[...] = emb_ref[...]


def kernel(x, emb):
    seq_len = x.shape[1]
    dim = emb.shape[1]
    num_blocks = seq_len // _BLOCK_ROWS
    return pl.pallas_call(
        _copy_body,
        grid=(num_blocks,),
        in_specs=[pl.BlockSpec((_BLOCK_ROWS, dim), lambda i: (i, 0))],
        out_specs=pl.BlockSpec((_BLOCK_ROWS, dim), lambda i: (i, 0)),
        out_shape=jax.ShapeDtypeStruct((seq_len, dim), emb.dtype),
    )(emb)


# speedup vs baseline: 6.1740x; 1.4050x over previous
import jax
import jax.numpy as jnp
from jax.experimental import pallas as pl

_BLOCK_ROWS = 2048


def _copy_body(emb_ref, o_ref):
    o_ref[...] = emb_ref[...]


def kernel(x, emb):
    seq_len = x.shape[1]
    dim = emb.shape[1]
    num_blocks = seq_len // _BLOCK_ROWS
    return pl.pallas_call(
        _copy_body,
        grid=(num_blocks,),
        in_specs=[pl.BlockSpec((_BLOCK_ROWS, dim), lambda i: (i, 0))],
        out_specs=pl.BlockSpec((_BLOCK_ROWS, dim), lambda i: (i, 0)),
        out_shape=jax.ShapeDtypeStruct((seq_len, dim), emb.dtype),
    )(emb)


# speedup vs baseline: 8.0939x; 1.3110x over previous
import jax
import jax.numpy as jnp
from jax.experimental import pallas as pl

_BLOCK_ROWS = 4096


def _copy_body(emb_ref, o_ref):
    o_ref[...] = emb_ref[...]


def kernel(x, emb):
    seq_len = x.shape[1]
    dim = emb.shape[1]
    num_blocks = seq_len // _BLOCK_ROWS
    return pl.pallas_call(
        _copy_body,
        grid=(num_blocks,),
        in_specs=[pl.BlockSpec((_BLOCK_ROWS, dim), lambda i: (i, 0))],
        out_specs=pl.BlockSpec((_BLOCK_ROWS, dim), lambda i: (i, 0)),
        out_shape=jax.ShapeDtypeStruct((seq_len, dim), emb.dtype),
    )(emb)
